# Initial kernel scaffold; baseline (speedup 1.0000x reference)
import functools

import jax
import jax.numpy as jnp
from jax import lax
from jax.experimental import pallas as pl
from jax.experimental.pallas import tpu as pltpu
from jax.experimental.pallas import tpu_sc as plsc

N_TOK = 8192
D_MODEL = 2048
NE = 5
TOK_TILE = 512


def _gate_body(x_ref, w1_ref, b1_ref, w2_ref, b2_ref, w3_ref, b3_ref,
               w4_ref, b4_ref, out_ref):
    h = jnp.maximum(
        jnp.dot(x_ref[...], w1_ref[...], preferred_element_type=jnp.float32)
        + b1_ref[...], 0.0)
    h = jnp.maximum(
        jnp.dot(h, w2_ref[...], preferred_element_type=jnp.float32)
        + b2_ref[...], 0.0)
    h = jnp.maximum(
        jnp.dot(h, w3_ref[...], preferred_element_type=jnp.float32)
        + b3_ref[...], 0.0)
    lg = jnp.dot(h, w4_ref[...], preferred_element_type=jnp.float32) + b4_ref[...]
    m = lg[:, 0:1]
    for k in range(1, NE):
        m = jnp.maximum(m, lg[:, k:k + 1])
    e = jnp.exp(lg - m)
    s = e[:, 0:1]
    for k in range(1, NE):
        s = s + e[:, k:k + 1]
    p = e / s
    lane = lax.broadcasted_iota(jnp.int32, p.shape, 1)
    out_ref[...] = jnp.where(lane < NE, p, 0.0)


def _gate_probs(x, W1, b1, W2, b2, W3, b3, W4, b4):
    W4p = jnp.zeros((128, 8), jnp.float32).at[:, :NE].set(W4)
    b4p = jnp.zeros((1, 8), jnp.float32).at[:, :NE].set(b4)
    grid = (N_TOK // TOK_TILE,)
    return pl.pallas_call(
        _gate_body,
        grid=grid,
        in_specs=[
            pl.BlockSpec((TOK_TILE, D_MODEL), lambda i: (i, 0)),
            pl.BlockSpec((D_MODEL, 1024), lambda i: (0, 0)),
            pl.BlockSpec((1, 1024), lambda i: (0, 0)),
            pl.BlockSpec((1024, 512), lambda i: (0, 0)),
            pl.BlockSpec((1, 512), lambda i: (0, 0)),
            pl.BlockSpec((512, 128), lambda i: (0, 0)),
            pl.BlockSpec((1, 128), lambda i: (0, 0)),
            pl.BlockSpec((128, 8), lambda i: (0, 0)),
            pl.BlockSpec((1, 8), lambda i: (0, 0)),
        ],
        out_specs=pl.BlockSpec((TOK_TILE, 8), lambda i: (i, 0)),
        out_shape=jax.ShapeDtypeStruct((N_TOK, 8), jnp.float32),
    )(x, W1, b1.reshape(1, -1), W2, b2.reshape(1, -1), W3,
      b3.reshape(1, -1), W4p, b4p)


def kernel(x, W1, b1, W2, b2, W3, b3, W4, b4):
    raise NotImplementedError


# speedup vs baseline: 68.5878x; 68.5878x over previous
import functools

import jax
import jax.numpy as jnp
import numpy as np
from jax import lax
from jax.experimental import pallas as pl
from jax.experimental.pallas import tpu as pltpu
from jax.experimental.pallas import tpu_sc as plsc

N_TOK = 8192
D_MODEL = 2048
NE = 5
MPE = 1024
CAP = MPE + (N_TOK - MPE * NE) // NE + 1
REM = N_TOK - NE * MPE
TOK_TILE = 256
NEG = np.int32(-2147483648)


def _gate_body(x_ref, w1_ref, b1_ref, w2_ref, b2_ref, w3_ref, b3_ref,
               w4_ref, b4_ref, out_ref):
    h = jnp.maximum(
        jnp.dot(x_ref[...], w1_ref[...], preferred_element_type=jnp.float32)
        + b1_ref[...], 0.0)
    h = jnp.maximum(
        jnp.dot(h, w2_ref[...], preferred_element_type=jnp.float32)
        + b2_ref[...], 0.0)
    h = jnp.maximum(
        jnp.dot(h, w3_ref[...], preferred_element_type=jnp.float32)
        + b3_ref[...], 0.0)
    out_ref[...] = (jnp.dot(h, w4_ref[...], preferred_element_type=jnp.float32)
                    + b4_ref[...])


def _gate_logits(x, W1, b1, W2, b2, W3, b3, W4, b4):
    W4p = jnp.zeros((128, 8), jnp.float32).at[:, :NE].set(W4)
    b4p = jnp.zeros((1, 8), jnp.float32).at[:, :NE].set(b4)
    return pl.pallas_call(
        _gate_body,
        grid=(N_TOK // TOK_TILE,),
        in_specs=[
            pl.BlockSpec((TOK_TILE, D_MODEL), lambda i: (i, 0)),
            pl.BlockSpec((D_MODEL, 1024), lambda i: (0, 0)),
            pl.BlockSpec((1, 1024), lambda i: (0, 0)),
            pl.BlockSpec((1024, 512), lambda i: (0, 0)),
            pl.BlockSpec((1, 512), lambda i: (0, 0)),
            pl.BlockSpec((512, 128), lambda i: (0, 0)),
            pl.BlockSpec((1, 128), lambda i: (0, 0)),
            pl.BlockSpec((128, 8), lambda i: (0, 0)),
            pl.BlockSpec((1, 8), lambda i: (0, 0)),
        ],
        out_specs=pl.BlockSpec((TOK_TILE, 8), lambda i: (i, 0)),
        out_shape=jax.ShapeDtypeStruct((N_TOK, 8), jnp.float32),
    )(x, W1, b1.reshape(1, -1), W2, b2.reshape(1, -1), W3,
      b3.reshape(1, -1), W4p, b4p)


_I16 = lambda: lax.iota(jnp.int32, 16)
_ONES = lambda: jnp.ones((16,), jnp.int32)


def _ext(vec, k):
    return jnp.max(jnp.where(_I16() == k, vec, NEG))


def _gat(vec, idx):
    return lax.gather(
        vec, idx.reshape(16, 1),
        lax.GatherDimensionNumbers(offset_dims=(), collapsed_slice_dims=(0,),
                                   start_index_map=(0,)),
        (1,), mode=lax.GatherScatterMode.PROMISE_IN_BOUNDS)


def _bc(s):
    return jnp.broadcast_to(s, (16,))


def _alloc_body(pbits_hbm, allidx_hbm, stats_hbm,
                pball, sidxb, ownr, slotv, cand, rlist,
                hist, cbase, cntv, c0v, statsv):
    wid = lax.axis_index("s") * 2 + lax.axis_index("c")

    @pl.when(wid == 0)
    def _main():
        i16 = _I16()
        SORTB = NE * N_TOK

        def fkk(kk, _):
            pltpu.sync_copy(pbits_hbm.at[pl.ds(kk * 16384, 16384)],
                            sidxb.at[pl.ds(0, 16384)])

            def fst(c, _):
                tl = c * 16 + _I16()
                for e in range(NE):
                    v = plsc.load_gather(sidxb, [tl * 8 + e])
                    pball[pl.ds(e * N_TOK + kk * 2048 + c * 16, 16)] = v
                return 0
            lax.fori_loop(0, 128, fst, 0)
            return 0
        lax.fori_loop(0, 4, fkk, 0)

        c0v[...] = jnp.zeros((16,), jnp.int32)

        def fch(c, _):
            t0 = c * 16
            best = pball[pl.ds(t0, 16)]
            ch = jnp.zeros((16,), jnp.int32)
            for e in range(1, NE):
                ve = pball[pl.ds(e * N_TOK + t0, 16)]
                m = ve > best
                best = jnp.where(m, ve, best)
                ch = jnp.where(m, jnp.int32(e), ch)
            ownr[pl.ds(t0, 16)] = ch
            plsc.addupdate_scatter(c0v, [ch], _ONES())
            return 0
        lax.fori_loop(0, N_TOK // 16, fch, 0)
        cntv[...] = c0v[...]

        def sort_col(o, _):
            c0_o = _ext(c0v[...], o)
            donor = c0_o > MPE
            recip = c0_o < MPE
            plane = o * N_TOK
            npass = jnp.where(donor, 4, jnp.where(recip, 9, 0))

            def run_pass(p, _):
                @pl.when(p < npass)
                def _run():
                    par = jnp.bitwise_and(p, 1)
                    d_src_sel = jnp.where(p == 0, 0, 2)
                    d_src_base = jnp.where(par == 1, SORTB, plane)
                    d_dst_base = jnp.where(par == 0, SORTB, plane)
                    r_src_sel = jnp.where(p == 0, 1, 2)
                    r_src_base = jnp.where(par == 0, SORTB, plane)
                    r_dst_base = jnp.where(par == 0, plane, SORTB)
                    r_mode = jnp.where(p < 4, 1, jnp.where(p == 4, 2, 0))
                    r_sh = jnp.where(p < 4, 8 * p, 8 * (p - 5))
                    src_sel = jnp.where(donor, d_src_sel, r_src_sel)
                    src_base = jnp.where(donor, d_src_base, r_src_base)
                    dst_base = jnp.where(donor, d_dst_base, r_dst_base)
                    mode = jnp.where(donor, 0, r_mode)
                    sh = jnp.where(donor, 8 * p, r_sh)

                    def get_tok(c):
                        it = c * 16 + _I16()
                        buf = sidxb[pl.ds(src_base + c * 16, 16)]
                        return jnp.where(
                            src_sel == 0, it,
                            jnp.where(src_sel == 1,
                                      jnp.int32(N_TOK - 1) - it, buf))

                    def byte_of(tok):
                        sc = plsc.load_gather(pball, [plane + tok])
                        och = plsc.load_gather(ownr, [tok])
                        ob = plsc.load_gather(pball, [och * N_TOK + tok])
                        inv = jnp.bitwise_xor(ob, jnp.int32(-1))
                        key = jnp.where(mode == 0, sc,
                                        jnp.where(mode == 1, inv,
                                                  jnp.int32(NE - 1) - och))
                        return jnp.bitwise_and(
                            lax.shift_right_logical(key, _bc(sh)), 255)

                    def fz(i, _):
                        hist[pl.ds(i * 16, 16)] = jnp.zeros((16,), jnp.int32)
                        return 0
                    lax.fori_loop(0, 16, fz, 0)

                    def fa(c, _):
                        plsc.addupdate_scatter(hist, [byte_of(get_tok(c))],
                                               _ONES())
                        return 0
                    lax.fori_loop(0, N_TOK // 16, fa, 0)

                    def fb(i, carry):
                        h = hist[pl.ds(i * 16, 16)]
                        cs = plsc.cumsum(h)
                        cbase[pl.ds(i * 16, 16)] = cs - h + carry
                        return carry + jnp.max(cs)
                    lax.fori_loop(0, 16, fb, jnp.int32(0))

                    def fc(c, _):
                        tok = get_tok(c)
                        b = byte_of(tok)
                        prior = plsc.scan_count(b)[0] - 1
                        pos = plsc.load_gather(cbase, [b]) + prior
                        plsc.store_scatter(sidxb, [dst_base + pos], tok)
                        plsc.addupdate_scatter(cbase, [b], _ONES())
                        return 0
                    lax.fori_loop(0, N_TOK // 16, fc, 0)
                return 0
            lax.fori_loop(0, 9, run_pass, 0)
            return 0
        lax.fori_loop(0, NE, sort_col, 0)

        def p1_col(e, _):
            need = MPE - _ext(c0v[...], e)

            @pl.when(need > 0)
            def _steal():
                tag = e + 1

                def mark_o(o, _):
                    ovf = _ext(cntv[...], o) - MPE

                    @pl.when(jnp.logical_and(o != e, ovf > 0))
                    def _mark():
                        def cond(st):
                            r, seen = st
                            return jnp.logical_and(r < N_TOK // 16,
                                                   seen < ovf)

                        def body(st):
                            r, seen = st
                            toks = sidxb[pl.ds(o * N_TOK + r * 16, 16)]
                            ow = plsc.load_gather(ownr, [toks])
                            m = ow == o
                            pc = plsc.cumsum(m.astype(jnp.int32))
                            cm = jnp.logical_and(m, (pc + seen) <= ovf)
                            plsc.store_scatter(cand, [toks], _bc(tag),
                                               mask=cm)
                            return r + 1, seen + jnp.max(pc)
                        lax.while_loop(cond, body,
                                       (jnp.int32(0), jnp.int32(0)))
                    return 0
                lax.fori_loop(0, NE, mark_o, 0)
                c0e = _ext(c0v[...], e)

                def cond(st):
                    r, got = st
                    return jnp.logical_and(r >= 0, got < need)

                def body(st):
                    r, got = st
                    toks = lax.rev(sidxb[pl.ds(e * N_TOK + r * 16, 16)],
                                   (0,))
                    cm = plsc.load_gather(cand, [toks]) == tag
                    pc = plsc.cumsum(cm.astype(jnp.int32))
                    pref = pc + got
                    sel = jnp.logical_and(cm, pref <= need)
                    plsc.store_scatter(slotv, [toks], _bc(c0e) + pref - 1,
                                       mask=sel)
                    ow_old = plsc.load_gather(ownr, [toks])
                    plsc.addupdate_scatter(cntv, [ow_old],
                                           jnp.full((16,), -1, jnp.int32),
                                           mask=sel)
                    plsc.store_scatter(ownr, [toks], _bc(e + 8), mask=sel)
                    return r - 1, got + jnp.max(pc)
                lax.while_loop(cond, body,
                               (jnp.int32(N_TOK // 16 - 1), jnp.int32(0)))
                plsc.addupdate_scatter(cntv, [_bc(e)], _bc(need),
                                       mask=(_I16() == 0))
            return 0
        lax.fori_loop(0, NE, p1_col, 0)

        ovfs = [jnp.maximum(_ext(cntv[...], e) - MPE, 0) for e in range(NE)]
        rbase = [jnp.int32(0)] * NE
        for e in range(1, NE):
            rbase[e] = rbase[e - 1] + ovfs[e - 1]

        def fmr(c, kr):
            t0 = c * 16
            toks = t0 + _I16()
            ow = ownr[pl.ds(t0, 16)]
            slc = slotv[pl.ds(t0, 16)]
            new_kr = []
            for e in range(NE):
                m = ow == e
                pc = plsc.cumsum(m.astype(jnp.int32))
                mrank = _bc(kr[e]) + pc - 1
                slc = jnp.where(m, mrank, slc)
                rmask = jnp.logical_and(m, mrank >= MPE)
                plsc.store_scatter(rlist, [_bc(rbase[e] - MPE) + mrank],
                                   toks, mask=rmask)
                new_kr.append(kr[e] + jnp.max(pc))
            slotv[pl.ds(t0, 16)] = slc
            return tuple(new_kr)
        lax.fori_loop(0, N_TOK // 16, fmr, (jnp.int32(0),) * NE)

        cnt0 = jnp.where(i16 < NE, jnp.int32(MPE), jnp.int32(CAP + 1024))

        def fp2(c, cnt):
            toks = rlist[pl.ds(c * 16, 16)]
            v = [plsc.load_gather(pball, [o * N_TOK + toks])
                 for o in range(NE)]
            cs = [_ext(cnt, o) for o in range(NE)]
            vms = [c_ < CAP for c_ in cs]
            any1 = vms[0]
            for o in range(1, NE):
                any1 = jnp.logical_or(any1, vms[o])
            best = jnp.full((16,), -1, jnp.int32)
            ch = jnp.zeros((16,), jnp.int32)
            for o in range(NE):
                allow = jnp.logical_or(vms[o], jnp.logical_not(any1))
                vo = jnp.where(allow, v[o], jnp.full((16,), -1, jnp.int32))
                m = vo > best
                best = jnp.where(m, vo, best)
                ch = jnp.where(m, jnp.int32(o), ch)
            prior = plsc.scan_count(ch)[0] - 1
            slotl = _gat(cnt, ch) + prior
            adds = [jnp.sum((ch == o).astype(jnp.int32)) for o in range(NE)]
            cnt_new = cnt
            for o in range(NE):
                cnt_new = jnp.where(i16 == o, cnt_new + adds[o], cnt_new)
            okall = (cs[0] + adds[0]) <= CAP
            for o in range(1, NE):
                okall = jnp.logical_and(okall, (cs[o] + adds[o]) <= CAP)
            valid = jnp.logical_or(okall, jnp.logical_not(any1))

            def fast(cnt=cnt, cnt_new=cnt_new, toks=toks, ch=ch,
                     slotl=slotl):
                plsc.store_scatter(ownr, [toks], ch)
                plsc.store_scatter(slotv, [toks], slotl)
                return cnt_new

            def slow(cnt=cnt, toks=toks, v=v):
                def fj(j, cc):
                    tv = _gat(toks, _bc(j))
                    ve = jnp.zeros((16,), jnp.int32)
                    for o in range(NE):
                        ve = jnp.where(_I16() == o, _gat(v[o], _bc(j)), ve)
                    vme = jnp.logical_and(cc < CAP, _I16() < NE)
                    anyv = jnp.max(vme.astype(jnp.int32)) > 0
                    vme2 = jnp.where(anyv, vme, _I16() < NE)
                    sb = jnp.where(vme2, ve, jnp.full((16,), -1, jnp.int32))
                    mx = jnp.max(sb)
                    chs = plsc.all_reduce_ffs(sb == mx)
                    chv = _bc(chs) if getattr(chs, "ndim", 1) == 0 else chs
                    slotj = _gat(cc, chv)
                    cc = cc + jnp.where(_I16() == chv, 1, 0).astype(jnp.int32)
                    lane0 = _I16() == 0
                    plsc.store_scatter(ownr, [tv], chv, mask=lane0)
                    plsc.store_scatter(slotv, [tv], slotj, mask=lane0)
                    return cc
                return lax.fori_loop(0, 16, fj, cnt)
            return lax.cond(valid, fast, slow)
        cntf = lax.fori_loop(0, REM // 16, fp2, cnt0)

        fcs = [_ext(cntf, e) for e in range(NE)]
        bases = [jnp.int32(0)] * NE
        for e in range(1, NE):
            bases[e] = bases[e - 1] + fcs[e - 1]
        basev = jnp.zeros((16,), jnp.int32)
        for e in range(NE):
            basev = jnp.where(i16 == e, _bc(bases[e]), basev)

        def fas(c, accs):
            t0 = c * 16
            toks = t0 + _I16()
            ow = jnp.bitwise_and(ownr[pl.ds(t0, 16)], 7)
            sl = slotv[pl.ds(t0, 16)]
            pos = _gat(basev, ow) + sl
            plsc.store_scatter(sidxb, [pos], toks)
            pbits = plsc.load_gather(pball, [ow * N_TOK + toks])
            pv = plsc.bitcast(pbits, jnp.float32)
            return tuple(accs[e] + jnp.where(ow == e, pv, 0.0)
                         for e in range(NE))
        accs = lax.fori_loop(0, N_TOK // 16, fas,
                             (jnp.zeros((16,), jnp.float32),) * NE)

        sv = jnp.zeros((16,), jnp.float32)
        for e in range(NE):
            sv = jnp.where(i16 == e, _bc(jnp.sum(accs[e])), sv)
            sv = jnp.where(i16 == NE + e, _bc(fcs[e].astype(jnp.float32)),
                           sv)
        statsv[...] = sv

        pltpu.sync_copy(sidxb.at[pl.ds(0, N_TOK)], allidx_hbm)
        pltpu.sync_copy(statsv, stats_hbm)


def _alloc(pbits):
    mesh = plsc.VectorSubcoreMesh(core_axis_name="c", subcore_axis_name="s")
    f = pl.kernel(
        _alloc_body,
        out_type=(jax.ShapeDtypeStruct((N_TOK,), jnp.int32),
                  jax.ShapeDtypeStruct((16,), jnp.float32)),
        mesh=mesh,
        scratch_types=[
            pltpu.VMEM((NE * N_TOK,), jnp.int32),
            pltpu.VMEM((NE * N_TOK + N_TOK,), jnp.int32),
            pltpu.VMEM((N_TOK,), jnp.int32),
            pltpu.VMEM((N_TOK,), jnp.int32),
            pltpu.VMEM((N_TOK,), jnp.int32),
            pltpu.VMEM((REM,), jnp.int32),
            pltpu.VMEM((256,), jnp.int32),
            pltpu.VMEM((256,), jnp.int32),
            pltpu.VMEM((16,), jnp.int32),
            pltpu.VMEM((16,), jnp.int32),
            pltpu.VMEM((16,), jnp.float32),
        ],
        compiler_params=pltpu.CompilerParams(needs_layout_passes=False),
    )
    return f(pbits)


def _gather_body(x_hbm, idx_hbm, out_hbm, idxv, rows, sem):
    wid = lax.axis_index("s") * 2 + lax.axis_index("c")
    base = wid * (N_TOK // 32)
    for k in range(N_TOK // 32 // 32):
        pltpu.sync_copy(idx_hbm.at[pl.ds(base + k * 32, 32)], idxv)
        pltpu.async_copy(x_hbm.at[idxv], rows, sem).wait()
        pltpu.sync_copy(rows, out_hbm.at[pl.ds(base + k * 32, 32)])


def _gather(x, all_idx):
    mesh = plsc.VectorSubcoreMesh(core_axis_name="c", subcore_axis_name="s")
    f = pl.kernel(
        _gather_body,
        out_type=jax.ShapeDtypeStruct((N_TOK, D_MODEL), jnp.float32),
        mesh=mesh,
        scratch_types=[
            pltpu.VMEM((32,), jnp.int32),
            pltpu.VMEM((32, D_MODEL), jnp.float32),
            pltpu.SemaphoreType.DMA,
        ],
    )
    return f(x, all_idx)


def kernel(x, W1, b1, W2, b2, W3, b3, W4, b4):
    logits = _gate_logits(x, W1, b1, W2, b2, W3, b3, W4, b4)
    probs = jax.nn.softmax(logits[:, :NE], axis=1)
    probs8 = jnp.concatenate(
        [probs, jnp.zeros((N_TOK, 8 - NE), jnp.float32)], axis=1)
    pbits = lax.bitcast_convert_type(probs8, jnp.int32).reshape(-1)
    all_idx, stats = _alloc(pbits)
    expert_data = _gather(x, all_idx)
    s = stats[0:NE]
    cf = stats[NE:2 * NE]
    mg = jnp.where(cf > 0, s / jnp.where(cf > 0, cf, 1.0),
                   jnp.float32(0.0))
    loss = jnp.sum(cf * mg) / NE
    return (expert_data, loss, all_idx)
